# baseline (speedup 1.0000x reference)
import functools

import numpy as np
import jax
import jax.numpy as jnp
from jax import lax
from jax.experimental import pallas as pl
from jax.experimental.pallas import tpu as pltpu
from jax.experimental.pallas import tpu_sc as plsc

B, N, D = 128, 64, 128
NS = 100
NB = 16
K = 16
RC = 2.0
UPDATE = 0.5
DECAY = 0.9
NCONV = 3
P_OUT = 64
E_OUT = 32

MB = 8
HI = lax.Precision.HIGHEST


CENTERS = (0.5, 0.6000000238418579, 0.7000000476837158, 0.800000011920929,
           0.9000000357627869, 1.0, 1.100000023841858, 1.2000000476837158,
           1.3000000715255737, 1.4000000953674316, 1.5, 1.600000023841858,
           1.7000000476837158, 1.8000000715255737, 1.9000000953674316, 2.0)


def _dot(a, b, precision=None):
    return lax.dot_general(a, b, (((1,), (0,)), ((), ())), precision=precision,
                           preferred_element_type=jnp.float32)


_SC_INFO = plsc.get_sparse_core_info()
_NW = _SC_INFO.num_cores * _SC_INFO.num_subcores
_BPW = (B * N) // _NW
_CH = 128
_NCH = _BPW // _CH


@functools.partial(
    pl.kernel,
    mesh=plsc.VectorSubcoreMesh(core_axis_name="c", subcore_axis_name="s"),
    out_type=jax.ShapeDtypeStruct((B * N, D), jnp.float32),
    scratch_types=[
        pltpu.VMEM((_NCH, _CH), jnp.int32),
        pltpu.VMEM((_BPW, D), jnp.float32),
        pltpu.SemaphoreType.DMA,
    ],
)
def _sc_embed(idx_hbm, table_hbm, out_hbm, idx_v, rows_v, sem):
    wid = lax.axis_index("s") * _SC_INFO.num_cores + lax.axis_index("c")
    pltpu.sync_copy(idx_hbm.at[pl.ds(wid * _NCH, _NCH)], idx_v)
    copies = [pltpu.async_copy(table_hbm.at[idx_v.at[c]],
                               rows_v.at[pl.ds(c * _CH, _CH)], sem)
              for c in range(_NCH)]
    for cp in copies:
        cp.wait()
    pltpu.sync_copy(rows_v, out_hbm.at[pl.ds(wid * _BPW, _BPW)])


def _mol_kernel(h0_ref, coords_ref, ct_ref, wrbf_ref, w1_ref,
                b1_ref, w2_ref, b2_ref, wp1_ref, bp1_ref, wp2_ref, bp2_ref,
                wq_ref, wk_ref, we1_ref, be1_ref, we2_ref, be2_ref,
                iso_ref, an_ref):
    f32 = jnp.float32

    h3 = h0_ref[...]
    hf = h3.reshape(MB * N, D)

    coords = coords_ref[...]
    ct = ct_ref[...]
    rx = coords[:, :, 0:1] - ct[:, 0:1, :]
    ry = coords[:, :, 1:2] - ct[:, 1:2, :]
    rz = coords[:, :, 2:3] - ct[:, 2:3, :]
    dist = jnp.sqrt(rx * rx + rz * rz + ry * ry + 1e-12)

    rbf_f = jnp.concatenate(
        [jnp.exp(-10.0 * (dist - np.float32(CENTERS[r])) ** 2)
         for r in range(NB)], axis=1)

    wrbf = wrbf_ref[...]
    w1 = w1_ref[...]; b1 = b1_ref[...]
    w2 = w2_ref[...]; b2 = b2_ref[...]

    for step in range(NCONV):
        h3 = hf.reshape(MB, N, D)
        agg = jnp.stack([_dot(rbf_f[mb], h3[mb]) for mb in range(MB)],
                        axis=0).reshape(MB, NB, N, D)
        ps = [agg[:, r] * wrbf[r:r + 1, :] for r in range(NB)]
        c = [ps[i] + ps[i + 8] for i in range(8)]
        s1 = [c[i] + c[i + 4] for i in range(4)]
        s2 = [s1[i] + s1[i + 2] for i in range(2)]
        m = s2[0] + s2[1]
        mf = m.reshape(MB * N, D)
        upd = _dot(jnp.tanh(_dot(mf, w1) + b1), w2) + b2
        hf = hf + (UPDATE * (DECAY ** step)) * upd

    h3 = hf.reshape(MB, N, D)

    pa = _dot(jnp.tanh(_dot(hf, wp1_ref[...]) + bp1_ref[...]),
              wp2_ref[...]) + bp2_ref[...]
    iso_ref[...] = pa.reshape(MB, N, P_OUT)

    q3 = _dot(hf, wq_ref[...]).reshape(MB, N, D)
    k3 = _dot(hf, wk_ref[...]).reshape(MB, N, D)
    scores = jnp.stack(
        [lax.dot_general(q3[mb], k3[mb], (((1,), (1,)), ((), ())),
                         preferred_element_type=f32)
         for mb in range(MB)], axis=0) / jnp.sqrt(f32(D))

    iota_i = lax.broadcasted_iota(jnp.int32, (MB, N, N), 1)
    iota_j = lax.broadcasted_iota(jnp.int32, (MB, N, N), 2)
    valid = (dist < RC) & (iota_i != iota_j)
    s = jnp.where(valid, scores, f32(-1e9))

    vals = []
    sels = []
    dsel = []
    rsel = []
    for _ in range(K):
        vmax = jnp.max(s, axis=-1, keepdims=True)
        cand = jnp.where(s == vmax, iota_j, N)
        amin = jnp.min(cand, axis=-1, keepdims=True)
        sel = iota_j == amin
        vals.append(vmax)
        sels.append(sel.astype(f32)[:, :, None, :])
        dsel.append(jnp.sum(jnp.where(sel, dist, 0.0), axis=-1, keepdims=True))
        rsel.append((jnp.sum(jnp.where(sel, rx, 0.0), axis=-1, keepdims=True),
                     jnp.sum(jnp.where(sel, ry, 0.0), axis=-1, keepdims=True),
                     jnp.sum(jnp.where(sel, rz, 0.0), axis=-1, keepdims=True)))
        s = jnp.where(sel, f32(-1e38), s)

    top_vals = jnp.concatenate(vals, axis=-1)
    mx = jnp.max(top_vals, axis=-1, keepdims=True)
    p = jnp.exp(top_vals - mx)
    attn = p / jnp.sum(p, axis=-1, keepdims=True)

    oh2f = jnp.concatenate(sels, axis=2).reshape(MB, N * K, N)
    hsel = jnp.stack([_dot(oh2f[mb], h3[mb], precision=HI) for mb in range(MB)],
                     axis=0).reshape(MB, N, K, D)

    pair = (h3[:, :, None, :] + hsel).reshape(MB * N * K, D)
    e = _dot(jnp.tanh(_dot(pair, we1_ref[...]) + be1_ref[...]),
             we2_ref[...]) + be2_ref[...]
    e4 = e.reshape(MB, N, K, E_OUT)

    uxs, uys, uzs, ats = [], [], [], []
    for t in range(K):
        den = dsel[t] + 1e-9
        sx, sy, sz = rsel[t]
        uxs.append((sx / den)[:, :, :, None])
        uys.append((sy / den)[:, :, :, None])
        uzs.append((sz / den)[:, :, :, None])
        ats.append(attn[:, :, t:t + 1][:, :, :, None])
    ux4 = jnp.concatenate(uxs, axis=2)
    uy4 = jnp.concatenate(uys, axis=2)
    uz4 = jnp.concatenate(uzs, axis=2)
    ea = e4 * jnp.concatenate(ats, axis=2)
    an_ref[...] = jnp.concatenate(
        [ux4 * ea, uy4 * ea, uz4 * ea], axis=-1)


def kernel(species, coords, emb_table, W_rbf, W1, b1, W2, b2, Wp1, bp1, Wp2,
           bp2, Wq, Wk, We1, be1, We2, be2):
    sp_chunks = species.astype(jnp.int32).reshape(B * N // _CH, _CH)
    h0 = _sc_embed(sp_chunks, emb_table).reshape(B, N, D)
    ct = jnp.swapaxes(coords, 1, 2)
    b1r = b1.reshape(1, D); b2r = b2.reshape(1, D)
    bp1r = bp1.reshape(1, D); bp2r = bp2.reshape(1, P_OUT)
    be1r = be1.reshape(1, D); be2r = be2.reshape(1, E_OUT)

    grid = (B // MB,)
    z2 = lambda i: (0, 0)
    in_specs = [
        pl.BlockSpec((MB, N, D), lambda i: (i, 0, 0)),
        pl.BlockSpec((MB, N, 3), lambda i: (i, 0, 0)),
        pl.BlockSpec((MB, 3, N), lambda i: (i, 0, 0)),
        pl.BlockSpec((NB, D), z2),
        pl.BlockSpec((D, D), z2), pl.BlockSpec((1, D), z2),
        pl.BlockSpec((D, D), z2), pl.BlockSpec((1, D), z2),
        pl.BlockSpec((D, D), z2), pl.BlockSpec((1, D), z2),
        pl.BlockSpec((D, P_OUT), z2), pl.BlockSpec((1, P_OUT), z2),
        pl.BlockSpec((D, D), z2),
        pl.BlockSpec((D, D), z2),
        pl.BlockSpec((D, D), z2), pl.BlockSpec((1, D), z2),
        pl.BlockSpec((D, E_OUT), z2), pl.BlockSpec((1, E_OUT), z2),
    ]
    out_specs = [
        pl.BlockSpec((MB, N, P_OUT), lambda i: (i, 0, 0)),
        pl.BlockSpec((MB, N, K, 3 * E_OUT), lambda i: (i, 0, 0, 0)),
    ]
    out_shape = [
        jax.ShapeDtypeStruct((B, N, P_OUT), jnp.float32),
        jax.ShapeDtypeStruct((B, N, K, 3 * E_OUT), jnp.float32),
    ]
    c_iso, an = pl.pallas_call(
        _mol_kernel,
        grid=grid,
        in_specs=in_specs,
        out_specs=out_specs,
        out_shape=out_shape,
        compiler_params=pltpu.CompilerParams(
            dimension_semantics=("arbitrary",)),
    )(h0, coords, ct, W_rbf, W1, b1r, W2, b2r, Wp1, bp1r, Wp2,
      bp2r, Wq, Wk, We1, be1r, We2, be2r)
    c_aniso = an.reshape(B, N, K * 3, E_OUT)
    return (c_iso, c_aniso)


# speedup vs baseline: 1.0172x; 1.0172x over previous
import functools

import numpy as np
import jax
import jax.numpy as jnp
from jax import lax
from jax.experimental import pallas as pl
from jax.experimental.pallas import tpu as pltpu
from jax.experimental.pallas import tpu_sc as plsc

B, N, D = 128, 64, 128
NS = 100
NB = 16
K = 16
RC = 2.0
UPDATE = 0.5
DECAY = 0.9
NCONV = 3
P_OUT = 64
E_OUT = 32

MB = 2
HI = lax.Precision.HIGHEST


CENTERS = (0.5, 0.6000000238418579, 0.7000000476837158, 0.800000011920929,
           0.9000000357627869, 1.0, 1.100000023841858, 1.2000000476837158,
           1.3000000715255737, 1.4000000953674316, 1.5, 1.600000023841858,
           1.7000000476837158, 1.8000000715255737, 1.9000000953674316, 2.0)


def _dot(a, b, precision=None):
    return lax.dot_general(a, b, (((1,), (0,)), ((), ())), precision=precision,
                           preferred_element_type=jnp.float32)


_SC_INFO = plsc.get_sparse_core_info()
_NW = _SC_INFO.num_cores * _SC_INFO.num_subcores
_BPW = (B * N) // _NW
_CH = 128
_NCH = _BPW // _CH


@functools.partial(
    pl.kernel,
    mesh=plsc.VectorSubcoreMesh(core_axis_name="c", subcore_axis_name="s"),
    out_type=jax.ShapeDtypeStruct((B * N, D), jnp.float32),
    scratch_types=[
        pltpu.VMEM((_NCH, _CH), jnp.int32),
        pltpu.VMEM((_BPW, D), jnp.float32),
        pltpu.SemaphoreType.DMA,
    ],
)
def _sc_embed(idx_hbm, table_hbm, out_hbm, idx_v, rows_v, sem):
    wid = lax.axis_index("s") * _SC_INFO.num_cores + lax.axis_index("c")
    pltpu.sync_copy(idx_hbm.at[pl.ds(wid * _NCH, _NCH)], idx_v)
    copies = [pltpu.async_copy(table_hbm.at[idx_v.at[c]],
                               rows_v.at[pl.ds(c * _CH, _CH)], sem)
              for c in range(_NCH)]
    for cp in copies:
        cp.wait()
    pltpu.sync_copy(rows_v, out_hbm.at[pl.ds(wid * _BPW, _BPW)])


def _mol_kernel(h0_ref, coords_ref, ct_ref, wrbf_ref, w1_ref,
                b1_ref, w2_ref, b2_ref, wp1_ref, bp1_ref, wp2_ref, bp2_ref,
                wq_ref, wk_ref, we1_ref, be1_ref, we2_ref, be2_ref,
                iso_ref, an_ref):
    f32 = jnp.float32

    h3 = h0_ref[...]
    hf = h3.reshape(MB * N, D)

    coords = coords_ref[...]
    ct = ct_ref[...]
    rx = coords[:, :, 0:1] - ct[:, 0:1, :]
    ry = coords[:, :, 1:2] - ct[:, 1:2, :]
    rz = coords[:, :, 2:3] - ct[:, 2:3, :]
    dist = jnp.sqrt(rx * rx + rz * rz + ry * ry + 1e-12)

    rbf_f = jnp.concatenate(
        [jnp.exp(-10.0 * (dist - np.float32(CENTERS[r])) ** 2)
         for r in range(NB)], axis=1)

    wrbf = wrbf_ref[...]
    w1 = w1_ref[...]; b1 = b1_ref[...]
    w2 = w2_ref[...]; b2 = b2_ref[...]

    for step in range(NCONV):
        h3 = hf.reshape(MB, N, D)
        agg = jnp.stack([_dot(rbf_f[mb], h3[mb]) for mb in range(MB)],
                        axis=0).reshape(MB, NB, N, D)
        ps = [agg[:, r] * wrbf[r:r + 1, :] for r in range(NB)]
        c = [ps[i] + ps[i + 8] for i in range(8)]
        s1 = [c[i] + c[i + 4] for i in range(4)]
        s2 = [s1[i] + s1[i + 2] for i in range(2)]
        m = s2[0] + s2[1]
        mf = m.reshape(MB * N, D)
        upd = _dot(jnp.tanh(_dot(mf, w1) + b1), w2) + b2
        hf = hf + (UPDATE * (DECAY ** step)) * upd

    h3 = hf.reshape(MB, N, D)

    pa = _dot(jnp.tanh(_dot(hf, wp1_ref[...]) + bp1_ref[...]),
              wp2_ref[...]) + bp2_ref[...]
    iso_ref[...] = pa.reshape(MB, N, P_OUT)

    q3 = _dot(hf, wq_ref[...]).reshape(MB, N, D)
    k3 = _dot(hf, wk_ref[...]).reshape(MB, N, D)
    scores = jnp.stack(
        [lax.dot_general(q3[mb], k3[mb], (((1,), (1,)), ((), ())),
                         preferred_element_type=f32)
         for mb in range(MB)], axis=0) / jnp.sqrt(f32(D))

    iota_i = lax.broadcasted_iota(jnp.int32, (MB, N, N), 1)
    iota_j = lax.broadcasted_iota(jnp.int32, (MB, N, N), 2)
    valid = (dist < RC) & (iota_i != iota_j)
    s = jnp.where(valid, scores, f32(-1e9))

    vals = []
    sels = []
    dsel = []
    rsel = []
    for _ in range(K):
        vmax = jnp.max(s, axis=-1, keepdims=True)
        cand = jnp.where(s == vmax, iota_j, N)
        amin = jnp.min(cand, axis=-1, keepdims=True)
        sel = iota_j == amin
        vals.append(vmax)
        sels.append(sel.astype(f32)[:, :, None, :])
        dsel.append(jnp.sum(jnp.where(sel, dist, 0.0), axis=-1, keepdims=True))
        rsel.append((jnp.sum(jnp.where(sel, rx, 0.0), axis=-1, keepdims=True),
                     jnp.sum(jnp.where(sel, ry, 0.0), axis=-1, keepdims=True),
                     jnp.sum(jnp.where(sel, rz, 0.0), axis=-1, keepdims=True)))
        s = jnp.where(sel, f32(-1e38), s)

    top_vals = jnp.concatenate(vals, axis=-1)
    mx = jnp.max(top_vals, axis=-1, keepdims=True)
    p = jnp.exp(top_vals - mx)
    attn = p / jnp.sum(p, axis=-1, keepdims=True)

    oh2f = jnp.concatenate(sels, axis=2).reshape(MB, N * K, N)
    hsel = jnp.stack([_dot(oh2f[mb], h3[mb], precision=HI) for mb in range(MB)],
                     axis=0).reshape(MB, N, K, D)

    pair = (h3[:, :, None, :] + hsel).reshape(MB * N * K, D)
    e = _dot(jnp.tanh(_dot(pair, we1_ref[...]) + be1_ref[...]),
             we2_ref[...]) + be2_ref[...]
    e4 = e.reshape(MB, N, K, E_OUT)

    uxs, uys, uzs, ats = [], [], [], []
    for t in range(K):
        den = dsel[t] + 1e-9
        sx, sy, sz = rsel[t]
        uxs.append((sx / den)[:, :, :, None])
        uys.append((sy / den)[:, :, :, None])
        uzs.append((sz / den)[:, :, :, None])
        ats.append(attn[:, :, t:t + 1][:, :, :, None])
    ux4 = jnp.concatenate(uxs, axis=2)
    uy4 = jnp.concatenate(uys, axis=2)
    uz4 = jnp.concatenate(uzs, axis=2)
    ea = e4 * jnp.concatenate(ats, axis=2)
    an_ref[...] = jnp.concatenate(
        [ux4 * ea, uy4 * ea, uz4 * ea], axis=-1)


def kernel(species, coords, emb_table, W_rbf, W1, b1, W2, b2, Wp1, bp1, Wp2,
           bp2, Wq, Wk, We1, be1, We2, be2):
    sp_chunks = species.astype(jnp.int32).reshape(B * N // _CH, _CH)
    h0 = _sc_embed(sp_chunks, emb_table).reshape(B, N, D)
    ct = jnp.swapaxes(coords, 1, 2)
    b1r = b1.reshape(1, D); b2r = b2.reshape(1, D)
    bp1r = bp1.reshape(1, D); bp2r = bp2.reshape(1, P_OUT)
    be1r = be1.reshape(1, D); be2r = be2.reshape(1, E_OUT)

    grid = (B // MB,)
    z2 = lambda i: (0, 0)
    in_specs = [
        pl.BlockSpec((MB, N, D), lambda i: (i, 0, 0)),
        pl.BlockSpec((MB, N, 3), lambda i: (i, 0, 0)),
        pl.BlockSpec((MB, 3, N), lambda i: (i, 0, 0)),
        pl.BlockSpec((NB, D), z2),
        pl.BlockSpec((D, D), z2), pl.BlockSpec((1, D), z2),
        pl.BlockSpec((D, D), z2), pl.BlockSpec((1, D), z2),
        pl.BlockSpec((D, D), z2), pl.BlockSpec((1, D), z2),
        pl.BlockSpec((D, P_OUT), z2), pl.BlockSpec((1, P_OUT), z2),
        pl.BlockSpec((D, D), z2),
        pl.BlockSpec((D, D), z2),
        pl.BlockSpec((D, D), z2), pl.BlockSpec((1, D), z2),
        pl.BlockSpec((D, E_OUT), z2), pl.BlockSpec((1, E_OUT), z2),
    ]
    out_specs = [
        pl.BlockSpec((MB, N, P_OUT), lambda i: (i, 0, 0)),
        pl.BlockSpec((MB, N, K, 3 * E_OUT), lambda i: (i, 0, 0, 0)),
    ]
    out_shape = [
        jax.ShapeDtypeStruct((B, N, P_OUT), jnp.float32),
        jax.ShapeDtypeStruct((B, N, K, 3 * E_OUT), jnp.float32),
    ]
    c_iso, an = pl.pallas_call(
        _mol_kernel,
        grid=grid,
        in_specs=in_specs,
        out_specs=out_specs,
        out_shape=out_shape,
        compiler_params=pltpu.CompilerParams(
            dimension_semantics=("arbitrary",)),
    )(h0, coords, ct, W_rbf, W1, b1r, W2, b2r, Wp1, bp1r, Wp2,
      bp2r, Wq, Wk, We1, be1r, We2, be2r)
    c_aniso = an.reshape(B, N, K * 3, E_OUT)
    return (c_iso, c_aniso)


# speedup vs baseline: 1.3414x; 1.3187x over previous
import functools

import numpy as np
import jax
import jax.numpy as jnp
from jax import lax
from jax.experimental import pallas as pl
from jax.experimental.pallas import tpu as pltpu
from jax.experimental.pallas import tpu_sc as plsc

B, N, D = 128, 64, 128
NS = 100
NB = 16
K = 16
RC = 2.0
UPDATE = 0.5
DECAY = 0.9
NCONV = 3
P_OUT = 64
E_OUT = 32

MB = 4
HI = lax.Precision.HIGHEST


CENTERS = (0.5, 0.6000000238418579, 0.7000000476837158, 0.800000011920929,
           0.9000000357627869, 1.0, 1.100000023841858, 1.2000000476837158,
           1.3000000715255737, 1.4000000953674316, 1.5, 1.600000023841858,
           1.7000000476837158, 1.8000000715255737, 1.9000000953674316, 2.0)


def _dot(a, b, precision=None):
    return lax.dot_general(a, b, (((1,), (0,)), ((), ())), precision=precision,
                           preferred_element_type=jnp.float32)


_SC_INFO = plsc.get_sparse_core_info()
_NW = _SC_INFO.num_cores * _SC_INFO.num_subcores
_BPW = (B * N) // _NW
_CH = 128
_NCH = _BPW // _CH


@functools.partial(
    pl.kernel,
    mesh=plsc.VectorSubcoreMesh(core_axis_name="c", subcore_axis_name="s"),
    out_type=jax.ShapeDtypeStruct((B * N, D), jnp.float32),
    scratch_types=[
        pltpu.VMEM((_NCH, _CH), jnp.int32),
        pltpu.VMEM((_BPW, D), jnp.float32),
        pltpu.SemaphoreType.DMA,
    ],
)
def _sc_embed(idx_hbm, table_hbm, out_hbm, idx_v, rows_v, sem):
    wid = lax.axis_index("s") * _SC_INFO.num_cores + lax.axis_index("c")
    pltpu.sync_copy(idx_hbm.at[pl.ds(wid * _NCH, _NCH)], idx_v)
    copies = [pltpu.async_copy(table_hbm.at[idx_v.at[c]],
                               rows_v.at[pl.ds(c * _CH, _CH)], sem)
              for c in range(_NCH)]
    for cp in copies:
        cp.wait()
    pltpu.sync_copy(rows_v, out_hbm.at[pl.ds(wid * _BPW, _BPW)])


def _mol_kernel(h0_ref, coords_ref, ct_ref, wrbf_ref, w1_ref,
                b1_ref, w2_ref, b2_ref, wp1_ref, bp1_ref, wp2_ref, bp2_ref,
                wq_ref, wk_ref, we1_ref, be1_ref, we2_ref, be2_ref,
                iso_ref, an_ref):
    f32 = jnp.float32

    h3 = h0_ref[...]
    hf = h3.reshape(MB * N, D)

    coords = coords_ref[...]
    ct = ct_ref[...]
    rx = coords[:, :, 0:1] - ct[:, 0:1, :]
    ry = coords[:, :, 1:2] - ct[:, 1:2, :]
    rz = coords[:, :, 2:3] - ct[:, 2:3, :]
    dist = jnp.sqrt(rx * rx + rz * rz + ry * ry + 1e-12)

    rbf_f = jnp.concatenate(
        [jnp.exp(-10.0 * (dist - np.float32(CENTERS[r])) ** 2)
         for r in range(NB)], axis=1)

    wrbf = wrbf_ref[...]
    w1 = w1_ref[...]; b1 = b1_ref[...]
    w2 = w2_ref[...]; b2 = b2_ref[...]

    for step in range(NCONV):
        h3 = hf.reshape(MB, N, D)
        agg = jnp.stack([_dot(rbf_f[mb], h3[mb]) for mb in range(MB)],
                        axis=0).reshape(MB, NB, N, D)
        ps = [agg[:, r] * wrbf[r:r + 1, :] for r in range(NB)]
        c = [ps[i] + ps[i + 8] for i in range(8)]
        s1 = [c[i] + c[i + 4] for i in range(4)]
        s2 = [s1[i] + s1[i + 2] for i in range(2)]
        m = s2[0] + s2[1]
        mf = m.reshape(MB * N, D)
        upd = _dot(jnp.tanh(_dot(mf, w1) + b1), w2) + b2
        hf = hf + (UPDATE * (DECAY ** step)) * upd

    h3 = hf.reshape(MB, N, D)

    pa = _dot(jnp.tanh(_dot(hf, wp1_ref[...]) + bp1_ref[...]),
              wp2_ref[...]) + bp2_ref[...]
    iso_ref[...] = pa.reshape(MB, N, P_OUT)

    q3 = _dot(hf, wq_ref[...]).reshape(MB, N, D)
    k3 = _dot(hf, wk_ref[...]).reshape(MB, N, D)
    scores = jnp.stack(
        [lax.dot_general(q3[mb], k3[mb], (((1,), (1,)), ((), ())),
                         preferred_element_type=f32)
         for mb in range(MB)], axis=0) / jnp.sqrt(f32(D))

    iota_i = lax.broadcasted_iota(jnp.int32, (MB, N, N), 1)
    iota_j = lax.broadcasted_iota(jnp.int32, (MB, N, N), 2)
    valid = (dist < RC) & (iota_i != iota_j)
    s = jnp.where(valid, scores, f32(-1e9))

    vals = []
    sels = []
    for _ in range(K):
        vmax = jnp.max(s, axis=-1, keepdims=True)
        cand = jnp.where(s == vmax, iota_j, N)
        amin = jnp.min(cand, axis=-1, keepdims=True)
        sel = iota_j == amin
        vals.append(vmax)
        sels.append(sel.astype(f32)[:, :, None, :])
        s = jnp.where(sel, f32(-1e38), s)

    top_vals = jnp.concatenate(vals, axis=-1)
    mx = jnp.max(top_vals, axis=-1, keepdims=True)
    p = jnp.exp(top_vals - mx)
    attn = p / jnp.sum(p, axis=-1, keepdims=True)

    oh2f = jnp.concatenate(sels, axis=2).reshape(MB, N * K, N)
    hsel = jnp.stack([_dot(oh2f[mb], h3[mb], precision=HI) for mb in range(MB)],
                     axis=0).reshape(MB, N, K, D)
    csel = jnp.stack([_dot(oh2f[mb], coords[mb], precision=HI)
                      for mb in range(MB)], axis=0).reshape(MB, N, K, 3)
    rsx4 = coords[:, :, None, 0:1] - csel[:, :, :, 0:1]
    rsy4 = coords[:, :, None, 1:2] - csel[:, :, :, 1:2]
    rsz4 = coords[:, :, None, 2:3] - csel[:, :, :, 2:3]
    den4 = jnp.sqrt(rsx4 * rsx4 + rsz4 * rsz4 + rsy4 * rsy4 + 1e-12) + 1e-9

    pair = (h3[:, :, None, :] + hsel).reshape(MB * N * K, D)
    e = _dot(jnp.tanh(_dot(pair, we1_ref[...]) + be1_ref[...]),
             we2_ref[...]) + be2_ref[...]
    e4 = e.reshape(MB, N, K, E_OUT)

    ats = [attn[:, :, t:t + 1][:, :, :, None] for t in range(K)]
    ea = e4 * jnp.concatenate(ats, axis=2)
    an_ref[...] = jnp.concatenate(
        [(rsx4 / den4) * ea, (rsy4 / den4) * ea, (rsz4 / den4) * ea],
        axis=-1)


def kernel(species, coords, emb_table, W_rbf, W1, b1, W2, b2, Wp1, bp1, Wp2,
           bp2, Wq, Wk, We1, be1, We2, be2):
    sp_chunks = species.astype(jnp.int32).reshape(B * N // _CH, _CH)
    h0 = _sc_embed(sp_chunks, emb_table).reshape(B, N, D)
    ct = jnp.swapaxes(coords, 1, 2)
    b1r = b1.reshape(1, D); b2r = b2.reshape(1, D)
    bp1r = bp1.reshape(1, D); bp2r = bp2.reshape(1, P_OUT)
    be1r = be1.reshape(1, D); be2r = be2.reshape(1, E_OUT)

    grid = (B // MB,)
    z2 = lambda i: (0, 0)
    in_specs = [
        pl.BlockSpec((MB, N, D), lambda i: (i, 0, 0)),
        pl.BlockSpec((MB, N, 3), lambda i: (i, 0, 0)),
        pl.BlockSpec((MB, 3, N), lambda i: (i, 0, 0)),
        pl.BlockSpec((NB, D), z2),
        pl.BlockSpec((D, D), z2), pl.BlockSpec((1, D), z2),
        pl.BlockSpec((D, D), z2), pl.BlockSpec((1, D), z2),
        pl.BlockSpec((D, D), z2), pl.BlockSpec((1, D), z2),
        pl.BlockSpec((D, P_OUT), z2), pl.BlockSpec((1, P_OUT), z2),
        pl.BlockSpec((D, D), z2),
        pl.BlockSpec((D, D), z2),
        pl.BlockSpec((D, D), z2), pl.BlockSpec((1, D), z2),
        pl.BlockSpec((D, E_OUT), z2), pl.BlockSpec((1, E_OUT), z2),
    ]
    out_specs = [
        pl.BlockSpec((MB, N, P_OUT), lambda i: (i, 0, 0)),
        pl.BlockSpec((MB, N, K, 3 * E_OUT), lambda i: (i, 0, 0, 0)),
    ]
    out_shape = [
        jax.ShapeDtypeStruct((B, N, P_OUT), jnp.float32),
        jax.ShapeDtypeStruct((B, N, K, 3 * E_OUT), jnp.float32),
    ]
    c_iso, an = pl.pallas_call(
        _mol_kernel,
        grid=grid,
        in_specs=in_specs,
        out_specs=out_specs,
        out_shape=out_shape,
        compiler_params=pltpu.CompilerParams(
            dimension_semantics=("arbitrary",)),
    )(h0, coords, ct, W_rbf, W1, b1r, W2, b2r, Wp1, bp1r, Wp2,
      bp2r, Wq, Wk, We1, be1r, We2, be2r)
    c_aniso = an.reshape(B, N, K * 3, E_OUT)
    return (c_iso, c_aniso)
